# baseline (speedup 1.0000x reference)
import jax
import jax.numpy as jnp
from jax import lax
from jax.experimental import pallas as pl
from jax.experimental.pallas import tpu as pltpu
from jax.experimental.pallas import tpu_sc as plsc

VOCAB = 1000000
DIM = 32
BATCH = 16384
HIST = 200

_NC = 2
_NS = 16
_NW = _NC * _NS

_P1_CHUNK = 1000
_P1_TOT = VOCAB // _P1_CHUNK

_BW = BATCH // _NW
_HGRP = 8
_NGRP = HIST // _HGRP


def _t_body(embT_hbm, trow_hbm, in_v, out_v):
    wid = lax.axis_index("s") * _NC + lax.axis_index("c")
    iota = lax.iota(jnp.int32, 16)
    iota_hi = iota + 16
    nchunk = (_P1_TOT - wid + _NW - 1) // _NW

    @pl.loop(0, nchunk)
    def _chunk(cc):
        col0 = (wid + cc * _NW) * _P1_CHUNK
        pltpu.sync_copy(embT_hbm.at[:, pl.ds(col0, _P1_CHUNK)], in_v)

        @pl.loop(0, _P1_CHUNK, unroll=8)
        def _col(j):
            jf = jnp.full((16,), j, jnp.int32)
            out_v[j, pl.ds(0, 16)] = plsc.load_gather(in_v, [iota, jf])
            out_v[j, pl.ds(16, 16)] = plsc.load_gather(in_v, [iota_hi, jf])

        pltpu.sync_copy(out_v, trow_hbm.at[pl.ds(col0, _P1_CHUNK), :])


def _g_body(trow_hbm, xT_hbm, out_hbm,
            idx_v, rows0, rows1, t0, t1,
            sem_g0, sem_g1, sem_o0, sem_o1):
    wid = lax.axis_index("s") * _NC + lax.axis_index("c")
    b0 = wid * _BW
    iota = lax.iota(jnp.int32, 16)
    iota_hi = iota + 16
    rows_v = [rows0, rows1]
    t_v = [t0, t1]
    sem_g = [sem_g0, sem_g1]
    sem_o = [sem_o0, sem_o1]

    def gather_copy(hh, p):
        return pltpu.make_async_copy(
            trow_hbm.at[idx_v.at[hh]], rows_v[p], sem_g[p]
        )

    def out_copy(h, p):
        return pltpu.make_async_copy(
            t_v[p], out_hbm.at[h, :, pl.ds(b0, _BW)], sem_o[p]
        )

    def transpose_block(p):
        rows = rows_v[p]
        t = t_v[p]

        @pl.loop(0, _BW, unroll=8)
        def _col(j):
            jf = jnp.full((16,), j, jnp.int32)
            plsc.store_scatter(t, [iota, jf], rows[j, pl.ds(0, 16)])
            plsc.store_scatter(t, [iota_hi, jf], rows[j, pl.ds(16, 16)])

    @pl.loop(0, _NGRP)
    def _grp(g):
        h0 = g * _HGRP
        pltpu.sync_copy(xT_hbm.at[pl.ds(h0, _HGRP), pl.ds(b0, _BW)], idx_v)
        gather_copy(0, 0).start()
        for hh in range(_HGRP):
            p = hh & 1
            gather_copy(hh, p).wait()
            if hh + 1 < _HGRP:
                gather_copy(hh + 1, 1 - p).start()
            if hh >= 2:
                out_copy(h0 + hh - 2, p).wait()
            transpose_block(p)
            out_copy(h0 + hh, p).start()
        out_copy(h0 + _HGRP - 2, 0).wait()
        out_copy(h0 + _HGRP - 1, 1).wait()


@jax.jit
def _run(x, embedding):
    xT = x.T
    embT = embedding.T

    mesh = plsc.VectorSubcoreMesh(
        core_axis_name="c", subcore_axis_name="s", num_cores=_NC
    )
    params = pltpu.CompilerParams(use_tc_tiling_on_sc=False, needs_layout_passes=False)

    transpose_k = pl.kernel(
        _t_body,
        out_type=jax.ShapeDtypeStruct((VOCAB, DIM), jnp.float32),
        mesh=mesh,
        scratch_types=[
            pltpu.VMEM((DIM, _P1_CHUNK), jnp.float32),
            pltpu.VMEM((_P1_CHUNK, DIM), jnp.float32),
        ],
        compiler_params=params,
    )
    trow = transpose_k(embT)

    gather_k = pl.kernel(
        _g_body,
        out_type=jax.ShapeDtypeStruct((HIST, DIM, BATCH), jnp.float32),
        mesh=mesh,
        scratch_types=[
            pltpu.VMEM((_HGRP, _BW), jnp.int32),
            pltpu.VMEM((_BW, DIM), jnp.float32),
            pltpu.VMEM((_BW, DIM), jnp.float32),
            pltpu.VMEM((DIM, _BW), jnp.float32),
            pltpu.VMEM((DIM, _BW), jnp.float32),
            pltpu.SemaphoreType.DMA,
            pltpu.SemaphoreType.DMA,
            pltpu.SemaphoreType.DMA,
            pltpu.SemaphoreType.DMA,
        ],
        compiler_params=params,
    )
    out_phys = gather_k(trow, xT)
    return out_phys.transpose(2, 0, 1)


def kernel(x, embedding):
    return _run(x, embedding)


# speedup vs baseline: 3.0624x; 3.0624x over previous
import jax
import jax.numpy as jnp
from jax import lax
from jax.experimental import pallas as pl
from jax.experimental.pallas import tpu as pltpu
from jax.experimental.pallas import tpu_sc as plsc

VOCAB = 1000000
DIM = 32
BATCH = 16384
HIST = 200

_NC = 2
_NS = 16
_NW = _NC * _NS

_TCW = 2048
_TGRID = (VOCAB + _TCW - 1) // _TCW

_BW = BATCH // _NW
_HGRP = 8
_NGRP = HIST // _HGRP


def _diag_perms():
    base = lax.iota(jnp.int32, 16)
    return base, base + 16, [jnp.bitwise_and(base + k, 15) for k in range(16)]


def _t_body(in_ref, out_ref):
    blk = in_ref[...]
    for q in range(4):
        out_ref[:, 32 * q:32 * (q + 1)] = blk[:, 512 * q:512 * (q + 1)].T


def _g_body(trow_hbm, xT_hbm, out_hbm,
            idx_v, rows0, rows1, t0, t1,
            sem_g0, sem_g1, sem_o0, sem_o1):
    wid = lax.axis_index("s") * _NC + lax.axis_index("c")
    b0 = wid * _BW
    iota, iota_hi, perms = _diag_perms()
    rows_v = [rows0, rows1]
    t_v = [t0, t1]
    sem_g = [sem_g0, sem_g1]
    sem_o = [sem_o0, sem_o1]

    def gather_copy(hh, p):
        return pltpu.make_async_copy(
            trow_hbm.at[idx_v.at[hh]], rows_v[p], sem_g[p]
        )

    def out_copy(h, p):
        return pltpu.make_async_copy(
            t_v[p],
            out_hbm.at[pl.ds(h * DIM, DIM), pl.ds(wid * (_BW // 128), _BW // 128), :],
            sem_o[p],
        )

    def transpose_block(p):
        rows = rows_v[p]
        t = t_v[p]

        @pl.loop(0, _BW // 16)
        def _blk(blk):
            j0 = blk * 16
            for k in range(16):
                jv = j0 + perms[k]
                jhi = jnp.right_shift(jv, 7)
                jlo = jnp.bitwise_and(jv, 127)
                v = plsc.load_gather(rows, [jv, iota])
                plsc.store_scatter(t, [iota, jhi, jlo], v)
                v = plsc.load_gather(rows, [jv, iota_hi])
                plsc.store_scatter(t, [iota_hi, jhi, jlo], v)

    @pl.loop(0, _NGRP)
    def _grp(g):
        h0 = g * _HGRP
        pltpu.sync_copy(xT_hbm.at[pl.ds(h0, _HGRP), pl.ds(b0, _BW)], idx_v)

        @pl.loop(0, _HGRP * _BW // 16)
        def _remap(m):
            hh = m // (_BW // 16)
            j0 = (m % (_BW // 16)) * 16
            v = idx_v[hh, pl.ds(j0, 16)]
            idx_v[hh, pl.ds(j0, 16)] = (
                jnp.bitwise_and(v, -2048)
                | jnp.left_shift(jnp.bitwise_and(v, 511), 2)
                | jnp.bitwise_and(jnp.right_shift(v, 9), 3)
            )

        gather_copy(0, 0).start()
        for hh in range(_HGRP):
            p = hh & 1
            gather_copy(hh, p).wait()
            if hh + 1 < _HGRP:
                gather_copy(hh + 1, 1 - p).start()
            if hh >= 2:
                out_copy(h0 + hh - 2, p).wait()
            transpose_block(p)
            out_copy(h0 + hh, p).start()
        out_copy(h0 + _HGRP - 2, 0).wait()
        out_copy(h0 + _HGRP - 1, 1).wait()


@jax.jit
def _run(x, embedding):
    xT = x.T
    embT = embedding.T

    trow_packed = pl.pallas_call(
        _t_body,
        grid=(_TGRID,),
        in_specs=[pl.BlockSpec((DIM, _TCW), lambda i: (0, i))],
        out_specs=pl.BlockSpec((_TCW // 4, 128), lambda i: (i, 0)),
        out_shape=jax.ShapeDtypeStruct((_TGRID * (_TCW // 4), 128), jnp.float32),
    )(embT)
    trow = trow_packed.reshape(_TGRID * _TCW, DIM)

    mesh = plsc.VectorSubcoreMesh(
        core_axis_name="c", subcore_axis_name="s", num_cores=_NC
    )
    params = pltpu.CompilerParams(
        use_tc_tiling_on_sc=False, needs_layout_passes=False
    )

    gather_k = pl.kernel(
        _g_body,
        out_type=jax.ShapeDtypeStruct((HIST * DIM, BATCH // 128, 128), jnp.float32),
        mesh=mesh,
        scratch_types=[
            pltpu.VMEM((_HGRP, _BW), jnp.int32),
            pltpu.VMEM((_BW, DIM), jnp.float32),
            pltpu.VMEM((_BW, DIM), jnp.float32),
            pltpu.VMEM((DIM, _BW // 128, 128), jnp.float32),
            pltpu.VMEM((DIM, _BW // 128, 128), jnp.float32),
            pltpu.SemaphoreType.DMA,
            pltpu.SemaphoreType.DMA,
            pltpu.SemaphoreType.DMA,
            pltpu.SemaphoreType.DMA,
        ],
        compiler_params=params,
    )
    out_packed = gather_k(trow, xT)
    out_phys = out_packed.reshape(HIST, DIM, BATCH)
    return out_phys.transpose(2, 0, 1)


def kernel(x, embedding):
    return _run(x, embedding)


# speedup vs baseline: 4.0223x; 1.3134x over previous
import jax
import jax.numpy as jnp
from jax import lax
from jax.experimental import pallas as pl
from jax.experimental.pallas import tpu as pltpu
from jax.experimental.pallas import tpu_sc as plsc

VOCAB = 1000000
DIM = 32
BATCH = 16384
HIST = 200

_NC = 2
_NS = 16
_NW = _NC * _NS

_TCW = 2048
_TGRID = (VOCAB + _TCW - 1) // _TCW

_BW = BATCH // _NW
_HGRP = 8
_NGRP = HIST // _HGRP


def _diag_perms():
    base = lax.iota(jnp.int32, 16)
    return base, base + 16, [jnp.bitwise_and(base + k, 15) for k in range(16)]


def _t_body(in_ref, out_ref):
    blk = in_ref[...]
    for q in range(4):
        out_ref[:, 32 * q:32 * (q + 1)] = blk[:, 512 * q:512 * (q + 1)].T


def _g_body(trow_hbm, xT_hbm, out_hbm,
            idx_v, rows0, rows1, t0, t1,
            sem_g0, sem_g1, sem_o0, sem_o1):
    wid = lax.axis_index("s") * _NC + lax.axis_index("c")
    b0 = wid * _BW
    iota, iota_hi, perms = _diag_perms()
    rows_v = [rows0, rows1]
    t_v = [t0, t1]
    sem_g = [sem_g0, sem_g1]
    sem_o = [sem_o0, sem_o1]

    def gather_copy(hh, p):
        return pltpu.make_async_copy(
            trow_hbm.at[idx_v.at[hh]], rows_v[p], sem_g[p]
        )

    def out_copy(h, p):
        return pltpu.make_async_copy(
            t_v[p],
            out_hbm.at[h, :, pl.ds(wid * (_BW // 128), _BW // 128), :, :],
            sem_o[p],
        )

    dt_lo = jnp.right_shift(iota, 3)
    din_lo = jnp.bitwise_and(iota, 7)
    dt_hi = jnp.right_shift(iota_hi, 3)
    din_hi = jnp.bitwise_and(iota_hi, 7)

    def transpose_block(p):
        rows = rows_v[p]
        t = t_v[p]

        @pl.loop(0, _BW // 16)
        def _blk(blk):
            j0 = blk * 16
            for k in range(16):
                jv = j0 + perms[k]
                jhi = jnp.right_shift(jv, 7)
                jlo = jnp.bitwise_and(jv, 127)
                v = plsc.load_gather(rows, [jv, iota])
                plsc.store_scatter(t, [dt_lo, jhi, din_lo, jlo], v)
                v = plsc.load_gather(rows, [jv, iota_hi])
                plsc.store_scatter(t, [dt_hi, jhi, din_hi, jlo], v)

    @pl.loop(0, _NGRP)
    def _grp(g):
        h0 = g * _HGRP
        pltpu.sync_copy(xT_hbm.at[pl.ds(h0, _HGRP), pl.ds(b0, _BW)], idx_v)

        @pl.loop(0, _HGRP * _BW // 16)
        def _remap(m):
            hh = m // (_BW // 16)
            j0 = (m % (_BW // 16)) * 16
            v = idx_v[hh, pl.ds(j0, 16)]
            idx_v[hh, pl.ds(j0, 16)] = (
                jnp.bitwise_and(v, -2048)
                | jnp.left_shift(jnp.bitwise_and(v, 511), 2)
                | jnp.bitwise_and(jnp.right_shift(v, 9), 3)
            )

        gather_copy(0, 0).start()
        for hh in range(_HGRP):
            p = hh & 1
            gather_copy(hh, p).wait()
            if hh + 1 < _HGRP:
                gather_copy(hh + 1, 1 - p).start()
            if hh >= 2:
                out_copy(h0 + hh - 2, p).wait()
            transpose_block(p)
            out_copy(h0 + hh, p).start()
        out_copy(h0 + _HGRP - 2, 0).wait()
        out_copy(h0 + _HGRP - 1, 1).wait()


@jax.jit
def _run(x, embedding):
    xT = x.T
    embT = embedding.T

    trow_packed = pl.pallas_call(
        _t_body,
        grid=(_TGRID,),
        in_specs=[pl.BlockSpec((DIM, _TCW), lambda i: (0, i))],
        out_specs=pl.BlockSpec((_TCW // 4, 128), lambda i: (i, 0)),
        out_shape=jax.ShapeDtypeStruct((_TGRID * (_TCW // 4), 128), jnp.float32),
    )(embT)
    trow = trow_packed.reshape(_TGRID * _TCW, DIM)

    mesh = plsc.VectorSubcoreMesh(
        core_axis_name="c", subcore_axis_name="s", num_cores=_NC
    )
    params = pltpu.CompilerParams(
        use_tc_tiling_on_sc=False, needs_layout_passes=False
    )

    gather_k = pl.kernel(
        _g_body,
        out_type=jax.ShapeDtypeStruct(
            (HIST, DIM // 8, BATCH // 128, 8, 128), jnp.float32
        ),
        mesh=mesh,
        scratch_types=[
            pltpu.VMEM((_HGRP, _BW), jnp.int32),
            pltpu.VMEM((_BW, DIM), jnp.float32),
            pltpu.VMEM((_BW, DIM), jnp.float32),
            pltpu.VMEM((DIM // 8, _BW // 128, 8, 128), jnp.float32),
            pltpu.VMEM((DIM // 8, _BW // 128, 8, 128), jnp.float32),
            pltpu.SemaphoreType.DMA,
            pltpu.SemaphoreType.DMA,
            pltpu.SemaphoreType.DMA,
            pltpu.SemaphoreType.DMA,
        ],
        compiler_params=params,
    )
    out_tiled = gather_k(trow, xT)
    return out_tiled.transpose(2, 4, 0, 1, 3).reshape(BATCH, HIST, DIM)


def kernel(x, embedding):
    return _run(x, embedding)


# speedup vs baseline: 4.5395x; 1.1286x over previous
import jax
import jax.numpy as jnp
from jax import lax
from jax.experimental import pallas as pl
from jax.experimental.pallas import tpu as pltpu
from jax.experimental.pallas import tpu_sc as plsc

VOCAB = 1000000
DIM = 32
BATCH = 16384
HIST = 200

_NC = 2
_NS = 16
_NW = _NC * _NS

_TCW = 4096
_TGRID = (VOCAB + _TCW - 1) // _TCW

_BW = BATCH // _NW
_HGRP = 20
_NGRP = HIST // _HGRP


def _diag_perms():
    base = lax.iota(jnp.int32, 16)
    return base, base + 16, [jnp.bitwise_and(base + k, 15) for k in range(16)]


def _t_body(in_ref, out_ref):
    blk = in_ref[...]
    for q in range(4):
        out_ref[:, 32 * q:32 * (q + 1)] = blk[:, 1024 * q:1024 * (q + 1)].T


def _g_body(trow_hbm, xT_hbm, out_hbm,
            idx_v, rows0, rows1, t0, t1,
            sem_g0, sem_g1, sem_o0, sem_o1):
    wid = lax.axis_index("s") * _NC + lax.axis_index("c")
    b0 = wid * _BW
    iota, iota_hi, perms = _diag_perms()
    rows_v = [rows0, rows1]
    t_v = [t0, t1]
    sem_g = [sem_g0, sem_g1]
    sem_o = [sem_o0, sem_o1]

    def gather_copy(hh, p):
        return pltpu.make_async_copy(
            trow_hbm.at[idx_v.at[hh]], rows_v[p], sem_g[p]
        )

    def out_copy(h, p):
        return pltpu.make_async_copy(
            t_v[p],
            out_hbm.at[h, :, pl.ds(wid * (_BW // 128), _BW // 128), :, :],
            sem_o[p],
        )

    dt_lo = jnp.right_shift(iota, 3)
    din_lo = jnp.bitwise_and(iota, 7)
    dt_hi = jnp.right_shift(iota_hi, 3)
    din_hi = jnp.bitwise_and(iota_hi, 7)

    def transpose_block(p):
        rows = rows_v[p]
        t = t_v[p]

        @pl.loop(0, _BW // 16)
        def _blk(blk):
            j0 = blk * 16
            jhi = jnp.full((16,), j0 >> 7, jnp.int32)
            jb = jnp.bitwise_and(j0, 127)
            for k in range(16):
                jv = j0 + perms[k]
                jlo = jb + perms[k]
                v = plsc.load_gather(rows, [jv, iota])
                plsc.store_scatter(t, [dt_lo, jhi, din_lo, jlo], v)
                v = plsc.load_gather(rows, [jv, iota_hi])
                plsc.store_scatter(t, [dt_hi, jhi, din_hi, jlo], v)

    @pl.loop(0, _NGRP)
    def _grp(g):
        h0 = g * _HGRP
        pltpu.sync_copy(xT_hbm.at[pl.ds(h0, _HGRP), pl.ds(b0, _BW)], idx_v)

        @pl.loop(0, _HGRP * _BW // 16)
        def _remap(m):
            hh = m // (_BW // 16)
            j0 = (m % (_BW // 16)) * 16
            v = idx_v[hh, pl.ds(j0, 16)]
            idx_v[hh, pl.ds(j0, 16)] = (
                jnp.bitwise_and(v, -4096)
                | jnp.left_shift(jnp.bitwise_and(v, 1023), 2)
                | jnp.bitwise_and(jnp.right_shift(v, 10), 3)
            )

        gather_copy(0, 0).start()
        for hh in range(_HGRP):
            p = hh & 1
            gather_copy(hh, p).wait()
            if hh + 1 < _HGRP:
                gather_copy(hh + 1, 1 - p).start()
            if hh >= 2:
                out_copy(h0 + hh - 2, p).wait()
            transpose_block(p)
            out_copy(h0 + hh, p).start()
        out_copy(h0 + _HGRP - 2, 0).wait()
        out_copy(h0 + _HGRP - 1, 1).wait()


@jax.jit
def _run(x, embedding):
    xT = x.T
    embT = embedding.T

    trow_packed = pl.pallas_call(
        _t_body,
        grid=(_TGRID,),
        in_specs=[pl.BlockSpec((DIM, _TCW), lambda i: (0, i))],
        out_specs=pl.BlockSpec((_TCW // 4, 128), lambda i: (i, 0)),
        out_shape=jax.ShapeDtypeStruct((_TGRID * (_TCW // 4), 128), jnp.float32),
    )(embT)
    trow = trow_packed.reshape(_TGRID * _TCW, DIM)

    mesh = plsc.VectorSubcoreMesh(
        core_axis_name="c", subcore_axis_name="s", num_cores=_NC
    )
    params = pltpu.CompilerParams(
        use_tc_tiling_on_sc=False, needs_layout_passes=False
    )

    gather_k = pl.kernel(
        _g_body,
        out_type=jax.ShapeDtypeStruct(
            (HIST, DIM // 8, BATCH // 128, 8, 128), jnp.float32
        ),
        mesh=mesh,
        scratch_types=[
            pltpu.VMEM((_HGRP, _BW), jnp.int32),
            pltpu.VMEM((_BW, DIM), jnp.float32),
            pltpu.VMEM((_BW, DIM), jnp.float32),
            pltpu.VMEM((DIM // 8, _BW // 128, 8, 128), jnp.float32),
            pltpu.VMEM((DIM // 8, _BW // 128, 8, 128), jnp.float32),
            pltpu.SemaphoreType.DMA,
            pltpu.SemaphoreType.DMA,
            pltpu.SemaphoreType.DMA,
            pltpu.SemaphoreType.DMA,
        ],
        compiler_params=params,
    )
    out_tiled = gather_k(trow, xT)
    return out_tiled.transpose(2, 4, 0, 1, 3).reshape(BATCH, HIST, DIM)


def kernel(x, embedding):
    return _run(x, embedding)
